# Initial kernel scaffold; baseline (speedup 1.0000x reference)
import jax
import jax.numpy as jnp
from jax.experimental import pallas as pl


def kernel(x, edge_index, n1, n2, W_self, b_self, W_conv, b_conv):
    raise NotImplementedError("write your pallas kernel here")


# speedup vs baseline: 19.1640x; 19.1640x over previous
import functools

import jax
import jax.numpy as jnp
from jax import lax
from jax.experimental import pallas as pl
from jax.experimental.pallas import tpu as pltpu
from jax.experimental.pallas import tpu_sc as plsc

N = 10000
NE = 320000
D = 128

NC = 2
NS = 16
NW = NC * NS

EPW = NE // NW

B = 128
NB = 79
SLOTS = NW * NB * B
ACC_ROWS = 10240
RPT = ACC_ROWS // NS
TRASH0 = N + 16
WB = N // NS

_MESH = plsc.VectorSubcoreMesh(core_axis_name="c", subcore_axis_name="s")


@functools.partial(
    pl.kernel,
    out_type=jax.ShapeDtypeStruct((NW, N), jnp.float32),
    mesh=_MESH,
    compiler_params=pltpu.CompilerParams(needs_layout_passes=False),
    scratch_types=[
        pltpu.VMEM((EPW,), jnp.int32),
        pltpu.VMEM((N,), jnp.float32),
    ],
)
def _deg_kernel(dst_hbm, part_hbm, idx_v, hist_v):
    cid = lax.axis_index("c")
    sid = lax.axis_index("s")
    wid = sid * NC + cid

    pltpu.sync_copy(dst_hbm.at[pl.ds(wid * EPW, EPW)], idx_v)

    zeros = jnp.zeros((16,), jnp.float32)

    def _zero(i, carry):
        hist_v[pl.ds(i * 16, 16)] = zeros
        return carry

    lax.fori_loop(0, N // 16, _zero, 0)

    ones = jnp.ones((16,), jnp.float32)

    def _scat(i, carry):
        idx = idx_v[pl.ds(i * 16, 16)]
        plsc.addupdate_scatter(hist_v, [idx], ones)
        return carry

    lax.fori_loop(0, EPW // 16, _scat, 0)

    pltpu.sync_copy(hist_v, part_hbm.at[wid])


def _dense_body(x_ref, ws_ref, wc_ref, bs_ref, bc_ref, degp_ref, hp_ref, base_ref):
    xb = x_ref[...]
    deg = jnp.sum(degp_ref[0], axis=0) + 1.0
    dinv = lax.rsqrt(deg)
    h = jnp.dot(xb, wc_ref[...], preferred_element_type=jnp.float32)
    hp = h * dinv[:, None]
    base = (
        jnp.dot(xb, ws_ref[...], preferred_element_type=jnp.float32)
        + bs_ref[...]
        + bc_ref[...]
        + hp * dinv[:, None]
    )
    hp_ref[...] = hp
    base_ref[...] = base


_R = 1000


def _dense(x, W_self, W_conv, b_self, b_conv, degp):
    return pl.pallas_call(
        _dense_body,
        grid=(N // _R,),
        in_specs=[
            pl.BlockSpec((_R, D), lambda i: (i, 0)),
            pl.BlockSpec((D, D), lambda i: (0, 0)),
            pl.BlockSpec((D, D), lambda i: (0, 0)),
            pl.BlockSpec((1, D), lambda i: (0, 0)),
            pl.BlockSpec((1, D), lambda i: (0, 0)),
            pl.BlockSpec((1, NW, _R), lambda i: (i, 0, 0)),
        ],
        out_specs=[
            pl.BlockSpec((_R, D), lambda i: (i, 0)),
            pl.BlockSpec((_R, D), lambda i: (i, 0)),
        ],
        out_shape=[
            jax.ShapeDtypeStruct((N, D), jnp.float32),
            jax.ShapeDtypeStruct((N, D), jnp.float32),
        ],
    )(x, W_self, W_conv, b_self, b_conv, degp)


@functools.partial(
    pl.kernel,
    out_type=jax.ShapeDtypeStruct((NC, ACC_ROWS, D), jnp.float32),
    mesh=_MESH,
    scratch_types=[
        pltpu.VMEM((NB, B), jnp.int32),
        pltpu.VMEM((NB, B), jnp.int32),
        pltpu.VMEM((B, D), jnp.float32),
        pltpu.VMEM((16, D), jnp.float32),
        pltpu.VMEM_SHARED((ACC_ROWS, D), jnp.float32),
        pltpu.SemaphoreType.DMA,
    ],
)
def _edge_kernel(hp_hbm, srcp_hbm, dstp_hbm, part_hbm,
                 src_v, dst_v, rows_v, zst_v, acc_sh, sem0):
    cid = lax.axis_index("c")
    sid = lax.axis_index("s")
    wid = sid * NC + cid

    zeros = jnp.zeros((16,), jnp.float32)

    def _zrow(i, carry):
        def _zcol(c, carry2):
            zst_v[i, pl.ds(c * 16, 16)] = zeros
            return carry2

        return lax.fori_loop(0, D // 16, _zcol, carry)

    lax.fori_loop(0, 16, _zrow, 0)

    def _zacc(j, carry):
        pltpu.sync_copy(zst_v, acc_sh.at[pl.ds(sid * RPT + j * 16, 16)])
        return carry

    lax.fori_loop(0, RPT // 16, _zacc, 0)

    pltpu.sync_copy(srcp_hbm.at[wid], src_v)
    pltpu.sync_copy(dstp_hbm.at[wid], dst_v)

    plsc.subcore_barrier()

    def _body(b, carry):
        pltpu.async_copy(hp_hbm.at[src_v.at[b]], rows_v, sem0).wait()
        pltpu.sync_copy(rows_v, acc_sh.at[dst_v.at[b]], add=True)
        return carry

    lax.fori_loop(0, NB, _body, 0)

    plsc.subcore_barrier()

    pltpu.sync_copy(acc_sh.at[pl.ds(sid * RPT, RPT)],
                    part_hbm.at[cid].at[pl.ds(sid * RPT, RPT)])


def _combine_body(base_ref, parts_ref, degp_ref, out_ref):
    deg = jnp.sum(degp_ref[0], axis=0) + 1.0
    dinv = lax.rsqrt(deg)
    s = parts_ref[0] + parts_ref[1]
    out_ref[...] = base_ref[...] + dinv[:, None] * s


def _combine(base, parts, degp):
    return pl.pallas_call(
        _combine_body,
        grid=(N // _R,),
        in_specs=[
            pl.BlockSpec((_R, D), lambda i: (i, 0)),
            pl.BlockSpec((NC, _R, D), lambda i: (0, i, 0)),
            pl.BlockSpec((1, NW, _R), lambda i: (i, 0, 0)),
        ],
        out_specs=pl.BlockSpec((_R, D), lambda i: (i, 0)),
        out_shape=jax.ShapeDtypeStruct((N, D), jnp.float32),
    )(base, parts, degp)


def kernel(x, edge_index, n1, n2, W_self, b_self, W_conv, b_conv):
    ei = edge_index.astype(jnp.int32)
    src = ei[0]
    dst = ei[1]

    degp = _deg_kernel(dst)
    degp_b = degp.reshape(NW, N // _R, _R).swapaxes(0, 1)

    hp, base = _dense(x, W_self, W_conv,
                      b_self.reshape(1, D), b_conv.reshape(1, D), degp_b)

    pad = SLOTS - NE
    src_p = jnp.concatenate([src, jnp.zeros((pad,), jnp.int32)]).reshape(NW, NB, B)
    trash = TRASH0 + (jnp.arange(pad, dtype=jnp.int32) % (ACC_ROWS - TRASH0))
    dst_p = jnp.concatenate([dst, trash]).reshape(NW, NB, B)

    parts = _edge_kernel(hp, src_p, dst_p)

    return _combine(base, parts, degp_b)
